# Initial kernel scaffold; baseline (speedup 1.0000x reference)
import jax
import jax.numpy as jnp
from jax.experimental import pallas as pl


def kernel(x, w, b):
    raise NotImplementedError("write your pallas kernel here")


# speedup vs baseline: 16.2840x; 16.2840x over previous
import functools

import jax
import jax.numpy as jnp
from jax import lax
from jax.experimental import pallas as pl
from jax.experimental.pallas import tpu as pltpu
from jax.experimental.pallas import tpu_sc as plsc

VSIZE = 100000
LENGTH = 200
BATCH = 1024
LANES = 16
LPAD = 208
NCHUNK = LPAD // LANES
NWORK = 32
CPW = BATCH // NWORK


def _nb2_body(xT_hbm, w_hbm, bvec_hbm, out_hbm,
              pos_v, idx_v, wv_v, bvec_v, pred_v, sem):
    lane = jnp.arange(LANES, dtype=jnp.int32)
    wid = lax.axis_index("s") * 2 + lax.axis_index("c")
    base = wid * CPW

    pltpu.sync_copy(xT_hbm.at[pl.ds(base, CPW)], idx_v)
    pltpu.sync_copy(bvec_hbm, bvec_v)

    cps = []
    for col in range(CPW):
        for off, sz in ((0, 128), (128, LPAD - 128)):
            cps.append(pltpu.async_copy(
                w_hbm.at[idx_v.at[col, pl.ds(off, sz)]],
                wv_v.at[col, pl.ds(off, sz)], sem))

    def init_body(i, carry):
        pos_v[pl.ds(i * LANES, LANES)] = jnp.full((LANES,), -1, jnp.int32)
        return carry

    lax.fori_loop(0, VSIZE // LANES, init_body, 0)

    for cp in cps:
        cp.wait()

    def col_body(col, carry):
        s_lo, s_hi = carry
        cbase = col * LPAD
        acc = jnp.zeros((LANES,), jnp.float32)
        for c in range(NCHUNK):
            toks = idx_v[col, pl.ds(c * LANES, LANES)]
            stamp = cbase + c * LANES + lane
            prev = plsc.load_gather(pos_v, [toks])
            fresh = prev < cbase
            if c == NCHUNK - 1:
                valid = lane < (LANES - (LPAD - LENGTH))
                plsc.store_scatter(pos_v, [toks], stamp, mask=valid)
            else:
                valid = None
                plsc.store_scatter(pos_v, [toks], stamp)
            post = plsc.load_gather(pos_v, [toks])
            keep = (post == stamp) & fresh
            if valid is not None:
                keep = keep & valid
            wv = wv_v[col, pl.ds(c * LANES, LANES)]
            acc = acc + jnp.where(keep, wv, 0.0)
        svec = jnp.full((LANES,), jnp.sum(acc))
        s_lo = jnp.where(lane == col, svec, s_lo)
        s_hi = jnp.where(lane == col - LANES, svec, s_hi)
        return s_lo, s_hi

    zeros = jnp.zeros((LANES,), jnp.float32)
    s_lo, s_hi = lax.fori_loop(0, CPW, col_body, (zeros, zeros))

    bv = bvec_v[...]
    pred_v[pl.ds(0, LANES)] = jnp.where(
        s_lo + bv > 0.0, 1, 0).astype(jnp.int32)
    pred_v[pl.ds(LANES, LANES)] = jnp.where(
        s_hi + bv > 0.0, 1, 0).astype(jnp.int32)
    pltpu.sync_copy(pred_v, out_hbm.at[pl.ds(base, CPW)])


def kernel(x, w, b):
    xT = jnp.pad(x.T.astype(jnp.int32), ((0, 0), (0, LPAD - LENGTH)))
    bvec = jnp.broadcast_to(b.astype(jnp.float32), (LANES,))
    run = functools.partial(
        pl.kernel,
        mesh=plsc.VectorSubcoreMesh(core_axis_name="c", subcore_axis_name="s"),
        out_type=jax.ShapeDtypeStruct((BATCH,), jnp.int32),
        compiler_params=pltpu.CompilerParams(needs_layout_passes=False),
        scratch_types=[
            pltpu.VMEM((VSIZE,), jnp.int32),
            pltpu.VMEM((CPW, LPAD), jnp.int32),
            pltpu.VMEM((CPW, LPAD), jnp.float32),
            pltpu.VMEM((LANES,), jnp.float32),
            pltpu.VMEM((NWORK,), jnp.int32),
            pltpu.SemaphoreType.DMA,
        ],
    )(_nb2_body)
    return run(xT, w.astype(jnp.float32), bvec)


# speedup vs baseline: 16.4963x; 1.0130x over previous
import functools

import jax
import jax.numpy as jnp
from jax import lax
from jax.experimental import pallas as pl
from jax.experimental.pallas import tpu as pltpu
from jax.experimental.pallas import tpu_sc as plsc

VSIZE = 100000
LENGTH = 200
BATCH = 1024
LANES = 16
LPAD = 208
NCHUNK = LPAD // LANES
NWORK = 32
CPW = BATCH // NWORK


def _nb2_body(xT_hbm, w_hbm, bvec_hbm, out_hbm,
              pos_v, idx_v, wv_v, bvec_v, pred_v, sem):
    lane = jnp.arange(LANES, dtype=jnp.int32)
    nvalid = LANES - (LPAD - LENGTH)
    wid = lax.axis_index("s") * 2 + lax.axis_index("c")
    base = wid * CPW

    pltpu.sync_copy(xT_hbm.at[pl.ds(base, CPW)], idx_v)
    pltpu.sync_copy(bvec_hbm, bvec_v)

    cps = []
    for col in range(CPW):
        for off, sz in ((0, 128), (128, LPAD - 128)):
            cps.append(pltpu.async_copy(
                w_hbm.at[idx_v.at[col, pl.ds(off, sz)]],
                wv_v.at[col, pl.ds(off, sz)], sem))

    for cp in cps:
        cp.wait()

    def col_body(col, carry):
        s_lo, s_hi = carry
        for c in range(NCHUNK):
            toks = idx_v[col, pl.ds(c * LANES, LANES)]
            stamp = c * LANES + lane
            if c == NCHUNK - 1:
                plsc.store_scatter(pos_v, [toks], stamp, mask=lane < nvalid)
            else:
                plsc.store_scatter(pos_v, [toks], stamp)
        acc = jnp.zeros((LANES,), jnp.float32)
        for c in range(NCHUNK):
            toks = idx_v[col, pl.ds(c * LANES, LANES)]
            stamp = c * LANES + lane
            post = plsc.load_gather(pos_v, [toks])
            keep = post == stamp
            if c == NCHUNK - 1:
                keep = keep & (lane < nvalid)
            wv = wv_v[col, pl.ds(c * LANES, LANES)]
            acc = acc + jnp.where(keep, wv, 0.0)
        svec = jnp.full((LANES,), jnp.sum(acc))
        s_lo = jnp.where(lane == col, svec, s_lo)
        s_hi = jnp.where(lane == col - LANES, svec, s_hi)
        return s_lo, s_hi

    zeros = jnp.zeros((LANES,), jnp.float32)
    s_lo, s_hi = lax.fori_loop(0, CPW, col_body, (zeros, zeros))

    bv = bvec_v[...]
    pred_v[pl.ds(0, LANES)] = jnp.where(
        s_lo + bv > 0.0, 1, 0).astype(jnp.int32)
    pred_v[pl.ds(LANES, LANES)] = jnp.where(
        s_hi + bv > 0.0, 1, 0).astype(jnp.int32)
    pltpu.sync_copy(pred_v, out_hbm.at[pl.ds(base, CPW)])


def kernel(x, w, b):
    xT = jnp.pad(x.T.astype(jnp.int32), ((0, 0), (0, LPAD - LENGTH)))
    bvec = jnp.broadcast_to(b.astype(jnp.float32), (LANES,))
    run = functools.partial(
        pl.kernel,
        mesh=plsc.VectorSubcoreMesh(core_axis_name="c", subcore_axis_name="s"),
        out_type=jax.ShapeDtypeStruct((BATCH,), jnp.int32),
        compiler_params=pltpu.CompilerParams(needs_layout_passes=False),
        scratch_types=[
            pltpu.VMEM((VSIZE,), jnp.int32),
            pltpu.VMEM((CPW, LPAD), jnp.int32),
            pltpu.VMEM((CPW, LPAD), jnp.float32),
            pltpu.VMEM((LANES,), jnp.float32),
            pltpu.VMEM((NWORK,), jnp.int32),
            pltpu.SemaphoreType.DMA,
        ],
    )(_nb2_body)
    return run(xT, w.astype(jnp.float32), bvec)


# speedup vs baseline: 24.9770x; 1.5141x over previous
import functools

import jax
import jax.numpy as jnp
from jax import lax
from jax.experimental import pallas as pl
from jax.experimental.pallas import tpu as pltpu
from jax.experimental.pallas import tpu_sc as plsc

VSIZE = 100000
LENGTH = 200
BATCH = 1024
LANES = 16
LPAD = 208
NCHUNK = LPAD // LANES
NWORK = 32
CPW = BATCH // NWORK
SLAB = CPW * LPAD


def _nb2_body(xT_hbm, w_hbm, bvec_hbm, out_hbm,
              wtab_v, idx_v, wsave_v, bvec_v, pred_v, sem):
    lane = jnp.arange(LANES, dtype=jnp.int32)
    nvalid = LANES - (LPAD - LENGTH)
    wid = lax.axis_index("s") * 2 + lax.axis_index("c")

    pltpu.sync_copy(xT_hbm.at[wid], idx_v)
    pltpu.sync_copy(w_hbm, wtab_v)
    pltpu.sync_copy(bvec_hbm, bvec_v)

    def col_body(col, carry):
        s_lo, s_hi = carry
        base = col * LPAD
        for c in range(NCHUNK):
            toks = idx_v[pl.ds(base + c * LANES, LANES)]
            wv = plsc.load_gather(wtab_v, [toks])
            wsave_v[pl.ds(c * LANES, LANES)] = wv
        for c in range(NCHUNK):
            toks = idx_v[pl.ds(base + c * LANES, LANES)]
            stamp = plsc.bitcast(c * LANES + lane, jnp.float32)
            if c == NCHUNK - 1:
                plsc.store_scatter(wtab_v, [toks], stamp, mask=lane < nvalid)
            else:
                plsc.store_scatter(wtab_v, [toks], stamp)
        acc = jnp.zeros((LANES,), jnp.float32)
        for c in range(NCHUNK):
            toks = idx_v[pl.ds(base + c * LANES, LANES)]
            post = plsc.bitcast(plsc.load_gather(wtab_v, [toks]), jnp.int32)
            keep = post == c * LANES + lane
            if c == NCHUNK - 1:
                keep = keep & (lane < nvalid)
            wv = wsave_v[pl.ds(c * LANES, LANES)]
            acc = acc + jnp.where(keep, wv, 0.0)
        for c in range(NCHUNK):
            toks = idx_v[pl.ds(base + c * LANES, LANES)]
            wv = wsave_v[pl.ds(c * LANES, LANES)]
            plsc.store_scatter(wtab_v, [toks], wv)
        svec = jnp.full((LANES,), jnp.sum(acc))
        s_lo = jnp.where(lane == col, svec, s_lo)
        s_hi = jnp.where(lane == col - LANES, svec, s_hi)
        return s_lo, s_hi

    zeros = jnp.zeros((LANES,), jnp.float32)
    s_lo, s_hi = lax.fori_loop(0, CPW, col_body, (zeros, zeros))

    bv = bvec_v[...]
    pred_v[pl.ds(0, LANES)] = jnp.where(
        s_lo + bv > 0.0, 1, 0).astype(jnp.int32)
    pred_v[pl.ds(LANES, LANES)] = jnp.where(
        s_hi + bv > 0.0, 1, 0).astype(jnp.int32)
    pltpu.sync_copy(pred_v, out_hbm.at[pl.ds(wid * CPW, CPW)])


def kernel(x, w, b):
    xT = jnp.pad(x.T.astype(jnp.int32), ((0, 0), (0, LPAD - LENGTH)))
    xT = xT.reshape(NWORK, SLAB)
    bvec = jnp.broadcast_to(b.astype(jnp.float32), (LANES,))
    run = functools.partial(
        pl.kernel,
        mesh=plsc.VectorSubcoreMesh(core_axis_name="c", subcore_axis_name="s"),
        out_type=jax.ShapeDtypeStruct((BATCH,), jnp.int32),
        compiler_params=pltpu.CompilerParams(needs_layout_passes=False),
        scratch_types=[
            pltpu.VMEM((VSIZE,), jnp.float32),
            pltpu.VMEM((SLAB,), jnp.int32),
            pltpu.VMEM((LPAD,), jnp.float32),
            pltpu.VMEM((LANES,), jnp.float32),
            pltpu.VMEM((NWORK,), jnp.int32),
            pltpu.SemaphoreType.DMA,
        ],
    )(_nb2_body)
    return run(xT, w.astype(jnp.float32), bvec)
